# baseline (speedup 1.0000x reference)
import functools

import jax
import jax.numpy as jnp
from jax import lax
from jax.experimental import pallas as pl
from jax.experimental.pallas import tpu as pltpu
from jax.experimental.pallas import tpu_sc as plsc

_L = 16
_W = 128


@functools.lru_cache(maxsize=None)
def _build_gather(hist, batch, vrows):
    info = plsc.get_sparse_core_info()
    nw = info.num_cores * info.num_subcores
    band = batch // nw
    nbb = band // _W
    nblk = hist * nbb
    assert batch % nw == 0 and band % _W == 0 and nblk % 2 == 0
    mesh = plsc.VectorSubcoreMesh(core_axis_name="c", subcore_axis_name="s")

    @functools.partial(
        pl.kernel,
        mesh=mesh,
        out_type=jax.ShapeDtypeStruct((hist, 64, batch), jnp.float32),
        scratch_types=[
            pltpu.VMEM((nblk * _W,), jnp.int32),
            pltpu.VMEM((nblk * _W,), jnp.int32),
            pltpu.VMEM((2, _W, _W), jnp.float32),
            pltpu.VMEM((2, 64, _W), jnp.float32),
            pltpu.SemaphoreType.DMA,
            pltpu.SemaphoreType.DMA,
            pltpu.SemaphoreType.DMA,
            pltpu.SemaphoreType.DMA,
            pltpu.SemaphoreType.DMA,
        ],
        compiler_params=pltpu.CompilerParams(
            use_tc_tiling_on_sc=True, needs_layout_passes=False
        ),
    )
    def gather_kernel(idx_hbm, table_hbm, out_hbm, dv, pv, rows, outv,
                      isem, gsem0, gsem1, osem0, osem1):
        gsems = (gsem0, gsem1)
        osems = (osem0, osem1)
        wid = lax.axis_index("s") * info.num_cores + lax.axis_index("c")
        col0 = wid * band
        iota = lax.iota(jnp.int32, _L)

        for h in range(hist):
            pltpu.async_copy(
                idx_hbm.at[h, pl.ds(col0, band)],
                dv.at[pl.ds(h * band, band)],
                isem,
            )
        for h in range(hist):
            pltpu.make_async_copy(
                idx_hbm.at[h, pl.ds(col0, band)],
                dv.at[pl.ds(h * band, band)],
                isem,
            ).wait()

        def prep(k, carry):
            v = dv[pl.ds(k * _L, _L)]
            pv[pl.ds(k * _L, _L)] = (v & 1) << 6
            dv[pl.ds(k * _L, _L)] = v >> 1
            return carry

        lax.fori_loop(0, nblk * _W // _L, prep, 0)

        def fire(i, slot):
            pltpu.async_copy(
                table_hbm.at[dv.at[pl.ds(i * _W, _W)]],
                rows.at[slot],
                gsems[slot],
            )

        def drain_gather(slot):
            pltpu.make_async_copy(
                table_hbm.at[dv.at[pl.ds(0, _W)]],
                rows.at[slot],
                gsems[slot],
            ).wait()

        def drain_out(i, slot):
            h = i // nbb
            c0 = col0 + (i % nbb) * _W
            for eb in range(8):
                pltpu.make_async_copy(
                    outv.at[slot].at[pl.ds(eb * 8, 8)],
                    out_hbm.at[h].at[pl.ds(eb * 8, 8), pl.ds(c0, _W)],
                    osems[slot],
                ).wait()

        fire(0, 0)

        def pair(p, carry):
            for b in range(2):
                i = p * 2 + b
                if b == 0:
                    fire(i + 1, 1)
                else:
                    @pl.when(i + 1 < nblk)
                    def _():
                        fire(i + 1, 0)
                drain_gather(b)

                @pl.when(i >= 2)
                def _():
                    drain_out(i - 2, b)

                for j in range(_W // _L):
                    rowsel = iota + (j * _L)
                    psel = pv[pl.ds(i * _W + j * _L, _L)]
                    for e in range(64):
                        val = plsc.load_gather(
                            rows.at[b], [rowsel, psel + e]
                        )
                        outv.at[b][e, pl.ds(j * _L, _L)] = val

                h = i // nbb
                c0 = col0 + (i % nbb) * _W
                for eb in range(8):
                    pltpu.async_copy(
                        outv.at[b].at[pl.ds(eb * 8, 8)],
                        out_hbm.at[h].at[pl.ds(eb * 8, 8), pl.ds(c0, _W)],
                        osems[b],
                    )
            return carry

        lax.fori_loop(0, nblk // 2, pair, 0)
        drain_out(nblk - 2, 0)
        drain_out(nblk - 1, 1)

    return gather_kernel


def kernel(word_indices, embeddings):
    batch, hist = word_indices.shape
    vocab, d = embeddings.shape
    assert d == 64
    table2 = embeddings.reshape(vocab // 2, 128)
    idx_t = word_indices.T
    out_phys = _build_gather(hist, batch, vocab // 2)(idx_t, table2)
    return out_phys.transpose(2, 0, 1)


# speedup vs baseline: 1.2724x; 1.2724x over previous
import functools

import jax
import jax.numpy as jnp
from jax import lax
from jax.experimental import pallas as pl
from jax.experimental.pallas import tpu as pltpu
from jax.experimental.pallas import tpu_sc as plsc

_L = 16
_W = 128
_WW = _W * _W


@functools.lru_cache(maxsize=None)
def _build_gather(hist, batch, vocab):
    info = plsc.get_sparse_core_info()
    nw = info.num_cores * info.num_subcores
    band = batch // nw
    nbb = band // _W
    nblk = hist * nbb
    assert batch % nw == 0 and band % _W == 0 and nblk % 2 == 0
    mesh = plsc.VectorSubcoreMesh(core_axis_name="c", subcore_axis_name="s")

    @functools.partial(
        pl.kernel,
        mesh=mesh,
        out_type=jax.ShapeDtypeStruct((hist, 64, batch), jnp.float32),
        scratch_types=[
            pltpu.VMEM((nblk * _W,), jnp.int32),
            pltpu.VMEM((2, _W, _W), jnp.float32),
            pltpu.VMEM((2, 64, _W), jnp.float32),
            pltpu.SemaphoreType.DMA,
            pltpu.SemaphoreType.DMA,
            pltpu.SemaphoreType.DMA,
            pltpu.SemaphoreType.DMA,
            pltpu.SemaphoreType.DMA,
        ],
        compiler_params=pltpu.CompilerParams(
            use_tc_tiling_on_sc=True, needs_layout_passes=False
        ),
    )
    def gather_kernel(idx_hbm, table_hbm, out_hbm, dv, rows, outv,
                      isem, gsem0, gsem1, osem0, osem1):
        gsems = (gsem0, gsem1)
        osems = (osem0, osem1)
        wid = lax.axis_index("s") * info.num_cores + lax.axis_index("c")
        col0 = wid * band
        iota = lax.iota(jnp.int32, _L)
        zero = iota * 0
        rowsels = [iota + (j * _L) for j in range(_W // _L)]

        for h in range(hist):
            pltpu.async_copy(
                idx_hbm.at[h, pl.ds(col0, band)],
                dv.at[pl.ds(h * band, band)],
                isem,
            )
        for h in range(hist):
            pltpu.make_async_copy(
                idx_hbm.at[h, pl.ds(col0, band)],
                dv.at[pl.ds(h * band, band)],
                isem,
            ).wait()

        def fire(i, slot):
            pltpu.async_copy(
                table_hbm.at[dv.at[pl.ds(i * _W, _W)]],
                rows.at[slot],
                gsems[slot],
            )

        def drain_gather(slot):
            pltpu.make_async_copy(
                table_hbm.at[dv.at[pl.ds(0, _W)]],
                rows.at[slot],
                gsems[slot],
            ).wait()

        def out_copies(i, slot, start):
            h = i // nbb
            c0 = col0 + (i % nbb) * _W
            for eb in range(8):
                cp = pltpu.make_async_copy(
                    outv.at[slot].at[pl.ds(eb * 8, 8)],
                    out_hbm.at[h].at[pl.ds(eb * 8, 8), pl.ds(c0, _W)],
                    osems[slot],
                )
                if start:
                    cp.start()
                else:
                    cp.wait()

        fire(0, 0)

        def pair(p, carry):
            for b in range(2):
                i = p * 2 + b
                if b == 0:
                    fire(i + 1, 1)
                else:
                    @pl.when(i + 1 < nblk)
                    def _():
                        fire(i + 1, 0)
                drain_gather(b)

                @pl.when(i >= 2)
                def _():
                    out_copies(i - 2, b, start=False)

                for e in range(64):
                    evec = zero + e
                    vals = [
                        plsc.load_gather(rows.at[b], [rowsels[j], evec])
                        for j in range(_W // _L)
                    ]
                    for j in range(_W // _L):
                        outv.at[b][e, pl.ds(j * _L, _L)] = vals[j]

                out_copies(i, b, start=True)
            return carry

        lax.fori_loop(0, nblk // 2, pair, 0)
        out_copies(nblk - 2, 0, start=False)
        out_copies(nblk - 1, 1, start=False)

    return gather_kernel


def kernel(word_indices, embeddings):
    batch, hist = word_indices.shape
    vocab, d = embeddings.shape
    assert d == 64
    table_p = jnp.pad(embeddings, ((0, 0), (0, 64)))
    idx_t = word_indices.T
    out_phys = _build_gather(hist, batch, vocab)(idx_t, table_p)
    return out_phys.transpose(2, 0, 1)


# speedup vs baseline: 1.8584x; 1.4605x over previous
import functools

import jax
import jax.numpy as jnp
from jax import lax
from jax.experimental import pallas as pl
from jax.experimental.pallas import tpu as pltpu
from jax.experimental.pallas import tpu_sc as plsc

_W = 128


@functools.lru_cache(maxsize=None)
def _build_gather(hist, batch, vocab):
    info = plsc.get_sparse_core_info()
    nw = info.num_cores * info.num_subcores
    band = batch // nw
    nbb = band // _W
    nblk = hist * nbb
    assert batch % nw == 0 and band % _W == 0 and nblk % 2 == 0
    mesh = plsc.VectorSubcoreMesh(core_axis_name="c", subcore_axis_name="s")

    @functools.partial(
        pl.kernel,
        mesh=mesh,
        out_type=jax.ShapeDtypeStruct((hist, batch, _W), jnp.float32),
        scratch_types=[
            pltpu.VMEM((nblk * _W,), jnp.int32),
            pltpu.VMEM((2, _W, _W), jnp.float32),
            pltpu.SemaphoreType.DMA,
            pltpu.SemaphoreType.DMA,
            pltpu.SemaphoreType.DMA,
            pltpu.SemaphoreType.DMA,
            pltpu.SemaphoreType.DMA,
        ],
        compiler_params=pltpu.CompilerParams(
            use_tc_tiling_on_sc=True, needs_layout_passes=False
        ),
    )
    def gather_kernel(idx_hbm, table_hbm, out_hbm, dv, rows,
                      isem, gsem0, gsem1, osem0, osem1):
        gsems = (gsem0, gsem1)
        osems = (osem0, osem1)
        wid = lax.axis_index("s") * info.num_cores + lax.axis_index("c")
        col0 = wid * band

        for h in range(hist):
            pltpu.async_copy(
                idx_hbm.at[h, pl.ds(col0, band)],
                dv.at[pl.ds(h * band, band)],
                isem,
            )
        for h in range(hist):
            pltpu.make_async_copy(
                idx_hbm.at[h, pl.ds(col0, band)],
                dv.at[pl.ds(h * band, band)],
                isem,
            ).wait()

        def fire(i, slot):
            pltpu.async_copy(
                table_hbm.at[dv.at[pl.ds(i * _W, _W)]],
                rows.at[slot],
                gsems[slot],
            )

        def drain_gather(slot):
            pltpu.make_async_copy(
                table_hbm.at[dv.at[pl.ds(0, _W)]],
                rows.at[slot],
                gsems[slot],
            ).wait()

        def out_copy(i, slot, start):
            h = i // nbb
            b0 = col0 + (i % nbb) * _W
            cp = pltpu.make_async_copy(
                rows.at[slot],
                out_hbm.at[h].at[pl.ds(b0, _W)],
                osems[slot],
            )
            if start:
                cp.start()
            else:
                cp.wait()

        fire(0, 0)

        def pair(p, carry):
            for b in range(2):
                i = p * 2 + b
                drain_gather(b)
                out_copy(i, b, start=True)
                if b == 0:
                    @pl.when(i >= 1)
                    def _():
                        out_copy(i - 1, 1, start=False)
                    fire(i + 1, 1)
                else:
                    out_copy(i - 1, 0, start=False)

                    @pl.when(i + 1 < nblk)
                    def _():
                        fire(i + 1, 0)
            return carry

        lax.fori_loop(0, nblk // 2, pair, 0)
        out_copy(nblk - 1, 1, start=False)

    return gather_kernel


def kernel(word_indices, embeddings):
    batch, hist = word_indices.shape
    vocab, d = embeddings.shape
    assert d == 64
    table_p = jnp.pad(embeddings, ((0, 0), (0, 64)))
    idx_t = word_indices.T
    lines = _build_gather(hist, batch, vocab)(idx_t, table_p)
    return lines.transpose(1, 0, 2)[:, :, :d]


# speedup vs baseline: 2.0201x; 1.0870x over previous
import functools

import jax
import jax.numpy as jnp
from jax import lax
from jax.experimental import pallas as pl
from jax.experimental.pallas import tpu as pltpu
from jax.experimental.pallas import tpu_sc as plsc

_W = 128


@functools.lru_cache(maxsize=None)
def _build_gather(hist, batch, vocab):
    info = plsc.get_sparse_core_info()
    nw = info.num_cores * info.num_subcores
    band = batch // nw
    nbb = band // _W
    nblk = hist * nbb
    assert batch % nw == 0 and band % _W == 0 and nblk % 2 == 0
    mesh = plsc.VectorSubcoreMesh(core_axis_name="c", subcore_axis_name="s")

    @functools.partial(
        pl.kernel,
        mesh=mesh,
        out_type=jax.ShapeDtypeStruct((hist, batch, _W), jnp.float32),
        scratch_types=[
            pltpu.VMEM((nblk * _W,), jnp.int32),
            pltpu.VMEM((4, _W, _W), jnp.float32),
            pltpu.SemaphoreType.DMA,
            pltpu.SemaphoreType.DMA,
            pltpu.SemaphoreType.DMA,
            pltpu.SemaphoreType.DMA,
            pltpu.SemaphoreType.DMA,
            pltpu.SemaphoreType.DMA,
            pltpu.SemaphoreType.DMA,
            pltpu.SemaphoreType.DMA,
            pltpu.SemaphoreType.DMA,
        ],
        compiler_params=pltpu.CompilerParams(
            use_tc_tiling_on_sc=True, needs_layout_passes=False
        ),
    )
    def gather_kernel(idx_hbm, table_hbm, out_hbm, dv, rows, isem,
                      gsem0, gsem1, gsem2, gsem3,
                      osem0, osem1, osem2, osem3):
        gsems = (gsem0, gsem1, gsem2, gsem3)
        osems = (osem0, osem1, osem2, osem3)
        wid = lax.axis_index("s") * info.num_cores + lax.axis_index("c")
        col0 = wid * band

        for h in range(hist):
            pltpu.async_copy(
                idx_hbm.at[h, pl.ds(col0, band)],
                dv.at[pl.ds(h * band, band)],
                isem,
            )
        for h in range(hist):
            pltpu.make_async_copy(
                idx_hbm.at[h, pl.ds(col0, band)],
                dv.at[pl.ds(h * band, band)],
                isem,
            ).wait()

        def fire(i, slot):
            pltpu.async_copy(
                table_hbm.at[dv.at[pl.ds(i * _W, _W)]],
                rows.at[slot],
                gsems[slot],
            )

        def drain_gather(slot):
            pltpu.make_async_copy(
                table_hbm.at[dv.at[pl.ds(0, _W)]],
                rows.at[slot],
                gsems[slot],
            ).wait()

        def out_copy(i, slot, start):
            h = i // nbb
            b0 = col0 + (i % nbb) * _W
            cp = pltpu.make_async_copy(
                rows.at[slot],
                out_hbm.at[h].at[pl.ds(b0, _W)],
                osems[slot],
            )
            if start:
                cp.start()
            else:
                cp.wait()

        fire(0, 0)
        fire(1, 1)

        def quad(p, carry):
            for b in range(4):
                i = p * 4 + b
                s = b
                ns = (b + 2) % 4
                drain_gather(s)
                out_copy(i, s, start=True)
                if b < 2:
                    @pl.when(i >= 2)
                    def _():
                        out_copy(i - 2, ns, start=False)
                    fire(i + 2, ns)
                else:
                    out_copy(i - 2, ns, start=False)

                    @pl.when(i + 2 < nblk)
                    def _():
                        fire(i + 2, ns)
            return carry

        lax.fori_loop(0, nblk // 4, quad, 0)
        out_copy(nblk - 2, (nblk - 2) % 4, start=False)
        out_copy(nblk - 1, (nblk - 1) % 4, start=False)

    return gather_kernel


def kernel(word_indices, embeddings):
    batch, hist = word_indices.shape
    vocab, d = embeddings.shape
    assert d == 64
    table_p = jnp.concatenate([embeddings, jnp.zeros((vocab, d), embeddings.dtype)], axis=1)
    idx_t = word_indices.T
    lines = _build_gather(hist, batch, vocab)(idx_t, table_p)
    return lines.transpose(1, 0, 2)[:, :, :d]
